# Initial kernel scaffold; baseline (speedup 1.0000x reference)
import jax
import jax.numpy as jnp
from jax.experimental import pallas as pl


def kernel(queries, keys):
    raise NotImplementedError("write your pallas kernel here")


# speedup vs baseline: 1.6047x; 1.6047x over previous
import functools

import jax
import jax.numpy as jnp
from jax.experimental import pallas as pl
from jax.experimental.pallas import tpu as pltpu

_BIG = 1e30


def _nn_kernel(q_ref, k_ref, o_ref, acc_ref, *, n_keys, kb, nk):
    j = pl.program_id(1)
    q = q_ref[...]
    k = k_ref[...]
    dots = jax.lax.dot_general(
        q, k, (((1,), (1,)), ((), ())),
        preferred_element_type=jnp.float32)
    ones = jnp.ones((8, q.shape[1]), jnp.float32)
    k_sq = jax.lax.dot_general(
        ones, k * k, (((1,), (1,)), ((), ())),
        preferred_element_type=jnp.float32)[:1]
    d = k_sq - 2.0 * dots
    cols = jax.lax.broadcasted_iota(jnp.int32, d.shape, 1) + j * kb
    d = jnp.where(cols < n_keys, d, _BIG)
    local = d[:, :128]
    for s in range(1, kb // 128):
        local = jnp.minimum(local, d[:, s * 128:(s + 1) * 128])

    @pl.when(j == 0)
    def _():
        acc_ref[...] = local

    @pl.when(j > 0)
    def _():
        acc_ref[...] = jnp.minimum(acc_ref[...], local)

    @pl.when(j == nk - 1)
    def _():
        q_sq = jnp.sum(q * q, axis=1, keepdims=True)
        m = jnp.min(acc_ref[...], axis=1, keepdims=True)
        o_ref[...] = jnp.sqrt(jnp.maximum(m + q_sq, 0.0) + 1e-12)


def kernel(queries, keys):
    n_q, d_dim = queries.shape
    n_k = keys.shape[0]
    qb = 256
    kb = 512
    nq = n_q // qb
    nk = (n_k + kb - 1) // kb
    out = pl.pallas_call(
        functools.partial(_nn_kernel, n_keys=n_k, kb=kb, nk=nk),
        grid=(nq, nk),
        in_specs=[
            pl.BlockSpec((qb, d_dim), lambda i, j: (i, 0)),
            pl.BlockSpec((kb, d_dim), lambda i, j: (j, 0)),
        ],
        out_specs=pl.BlockSpec((qb, 1), lambda i, j: (i, 0)),
        out_shape=jax.ShapeDtypeStruct((n_q, 1), jnp.float32),
        scratch_shapes=[pltpu.VMEM((qb, 128), jnp.float32)],
        compiler_params=pltpu.CompilerParams(
            dimension_semantics=("parallel", "arbitrary")),
    )(queries, keys)
    return out[:, 0]


# speedup vs baseline: 1.6278x; 1.0144x over previous
import functools

import jax
import jax.numpy as jnp
from jax.experimental import pallas as pl
from jax.experimental.pallas import tpu as pltpu

_BIG = 1e30


def _nn_kernel(q_ref, k_ref, o_ref, acc_ref, *, n_keys, kb, nk):
    j = pl.program_id(1)
    q = q_ref[...]
    k = k_ref[...]
    dots = jax.lax.dot_general(
        q.astype(jnp.bfloat16), k.astype(jnp.bfloat16),
        (((1,), (1,)), ((), ())),
        preferred_element_type=jnp.float32)
    ones = jnp.ones((8, q.shape[1]), jnp.float32)
    k_sq = jax.lax.dot_general(
        ones, k * k, (((1,), (1,)), ((), ())),
        preferred_element_type=jnp.float32)[:1]
    d = k_sq - 2.0 * dots

    def fold(x):
        local = x[:, :128]
        for s in range(1, kb // 128):
            local = jnp.minimum(local, x[:, s * 128:(s + 1) * 128])
        return local

    @pl.when(j == 0)
    def _():
        acc_ref[...] = jnp.full(acc_ref.shape, _BIG, jnp.float32)

    @pl.when(j < nk - 1)
    def _():
        acc_ref[...] = jnp.minimum(acc_ref[...], fold(d))

    @pl.when(j == nk - 1)
    def _():
        cols = jax.lax.broadcasted_iota(jnp.int32, d.shape, 1) + j * kb
        dm = jnp.where(cols < n_keys, d, _BIG)
        acc = jnp.minimum(acc_ref[...], fold(dm))
        q_sq = jnp.sum(q * q, axis=1, keepdims=True)
        m = jnp.min(acc, axis=1, keepdims=True)
        o_ref[...] = jnp.sqrt(jnp.maximum(m + q_sq, 0.0) + 1e-12)


def kernel(queries, keys):
    n_q, d_dim = queries.shape
    n_k = keys.shape[0]
    qb = 256
    kb = 512
    nq = n_q // qb
    nk = (n_k + kb - 1) // kb
    out = pl.pallas_call(
        functools.partial(_nn_kernel, n_keys=n_k, kb=kb, nk=nk),
        grid=(nq, nk),
        in_specs=[
            pl.BlockSpec((qb, d_dim), lambda i, j: (i, 0)),
            pl.BlockSpec((kb, d_dim), lambda i, j: (j, 0)),
        ],
        out_specs=pl.BlockSpec((qb, 1), lambda i, j: (i, 0)),
        out_shape=jax.ShapeDtypeStruct((n_q, 1), jnp.float32),
        scratch_shapes=[pltpu.VMEM((qb, 128), jnp.float32)],
        compiler_params=pltpu.CompilerParams(
            dimension_semantics=("parallel", "arbitrary")),
    )(queries, keys)
    return out[:, 0]


# speedup vs baseline: 5.6244x; 3.4552x over previous
import functools

import jax
import jax.numpy as jnp
from jax.experimental import pallas as pl
from jax.experimental.pallas import tpu as pltpu

_BIG = 1e30


def _nn_kernel(q_ref, k_ref, o_ref, acc_ref, *, n_keys, kb, nk):
    j = pl.program_id(1)
    q = q_ref[...]
    k = k_ref[...]
    dots = jax.lax.dot_general(
        q.astype(jnp.bfloat16), k.astype(jnp.bfloat16),
        (((1,), (1,)), ((), ())),
        preferred_element_type=jnp.float32)
    ones = jnp.ones((8, q.shape[1]), jnp.float32)
    k_sq = jax.lax.dot_general(
        ones, k * k, (((1,), (1,)), ((), ())),
        preferred_element_type=jnp.float32)[:1]
    d = k_sq - 2.0 * dots

    def fold(x):
        local = x[:, :128]
        for s in range(1, kb // 128):
            local = jnp.minimum(local, x[:, s * 128:(s + 1) * 128])
        return local

    @pl.when(j == 0)
    def _():
        acc_ref[...] = jnp.full(acc_ref.shape, _BIG, jnp.float32)

    @pl.when(j < nk - 1)
    def _():
        acc_ref[...] = jnp.minimum(acc_ref[...], fold(d))

    @pl.when(j == nk - 1)
    def _():
        cols = jax.lax.broadcasted_iota(jnp.int32, d.shape, 1) + j * kb
        dm = jnp.where(cols < n_keys, d, _BIG)
        acc = jnp.minimum(acc_ref[...], fold(dm))
        q_sq = jnp.sum(q * q, axis=1, keepdims=True)
        m = jnp.min(acc, axis=1, keepdims=True)
        o_ref[...] = jnp.sqrt(jnp.maximum(m + q_sq, 0.0) + 1e-12)


def kernel(queries, keys):
    n_q, d_dim = queries.shape
    n_k = keys.shape[0]
    qb = 1024
    kb = 1024
    nq = n_q // qb
    nk = (n_k + kb - 1) // kb
    out = pl.pallas_call(
        functools.partial(_nn_kernel, n_keys=n_k, kb=kb, nk=nk),
        grid=(nq, nk),
        in_specs=[
            pl.BlockSpec((qb, d_dim), lambda i, j: (i, 0)),
            pl.BlockSpec((kb, d_dim), lambda i, j: (j, 0)),
        ],
        out_specs=pl.BlockSpec((qb, 1), lambda i, j: (i, 0)),
        out_shape=jax.ShapeDtypeStruct((n_q, 1), jnp.float32),
        scratch_shapes=[pltpu.VMEM((qb, 128), jnp.float32)],
        compiler_params=pltpu.CompilerParams(
            dimension_semantics=("parallel", "arbitrary")),
    )(queries, keys)
    return out[:, 0]


# speedup vs baseline: 5.7881x; 1.0291x over previous
import functools

import jax
import jax.numpy as jnp
from jax.experimental import pallas as pl
from jax.experimental.pallas import tpu as pltpu

_BIG = 1e30


def _nn_kernel(q_ref, k_ref, o_ref, acc_ref, *, n_keys, kb, nk):
    j = pl.program_id(1)
    q = q_ref[...]
    k = k_ref[...]
    dots = jax.lax.dot_general(
        q.astype(jnp.bfloat16), k.astype(jnp.bfloat16),
        (((1,), (1,)), ((), ())),
        preferred_element_type=jnp.float32)
    halves = jnp.full((8, q.shape[1]), 0.5, jnp.float32)
    half_ksq = jax.lax.dot_general(
        halves, k * k, (((1,), (1,)), ((), ())),
        preferred_element_type=jnp.float32)[:1]
    d = half_ksq - dots

    def fold(x):
        local = x[:, :128]
        for s in range(1, kb // 128):
            local = jnp.minimum(local, x[:, s * 128:(s + 1) * 128])
        return local

    @pl.when(j == 0)
    def _():
        acc_ref[...] = jnp.full(acc_ref.shape, _BIG, jnp.float32)

    @pl.when(j < nk - 1)
    def _():
        acc_ref[...] = jnp.minimum(acc_ref[...], fold(d))

    @pl.when(j == nk - 1)
    def _():
        cols = jax.lax.broadcasted_iota(jnp.int32, d.shape, 1) + j * kb
        dm = jnp.where(cols < n_keys, d, _BIG)
        acc = jnp.minimum(acc_ref[...], fold(dm))
        q_sq = jnp.sum(q * q, axis=1, keepdims=True)
        m = jnp.min(acc, axis=1, keepdims=True)
        o_ref[...] = jnp.sqrt(jnp.maximum(2.0 * m + q_sq, 0.0) + 1e-12)


def kernel(queries, keys):
    n_q, d_dim = queries.shape
    n_k = keys.shape[0]
    qb = 512
    kb = 2048
    nq = n_q // qb
    nk = (n_k + kb - 1) // kb
    out = pl.pallas_call(
        functools.partial(_nn_kernel, n_keys=n_k, kb=kb, nk=nk),
        grid=(nq, nk),
        in_specs=[
            pl.BlockSpec((qb, d_dim), lambda i, j: (i, 0)),
            pl.BlockSpec((kb, d_dim), lambda i, j: (j, 0)),
        ],
        out_specs=pl.BlockSpec((qb, 1), lambda i, j: (i, 0)),
        out_shape=jax.ShapeDtypeStruct((n_q, 1), jnp.float32),
        scratch_shapes=[pltpu.VMEM((qb, 128), jnp.float32)],
        compiler_params=pltpu.CompilerParams(
            dimension_semantics=("parallel", "arbitrary")),
    )(queries, keys)
    return out[:, 0]


# speedup vs baseline: 5.8751x; 1.0150x over previous
import functools

import jax
import jax.numpy as jnp
from jax.experimental import pallas as pl
from jax.experimental.pallas import tpu as pltpu

_BIG = 1e30


def _nn_kernel(q_ref, k_ref, o_ref, acc_ref, *, n_keys, kb, nk, cb):
    j = pl.program_id(1)
    q = q_ref[...]
    k = k_ref[...]
    qbf = q.astype(jnp.bfloat16)
    halves = jnp.full((8, q.shape[1]), 0.5, jnp.float32)

    def fold(masked):
        local = None
        for c in range(kb // cb):
            kc = k if cb == kb else k[c * cb:(c + 1) * cb]
            dots = jax.lax.dot_general(
                qbf, kc.astype(jnp.bfloat16), (((1,), (1,)), ((), ())),
                preferred_element_type=jnp.float32)
            half_ksq = jax.lax.dot_general(
                halves, kc * kc, (((1,), (1,)), ((), ())),
                preferred_element_type=jnp.float32)[:1]
            for s in range(cb // 128):
                sl = slice(s * 128, (s + 1) * 128)
                ds = half_ksq[:, sl] - dots[:, sl]
                if masked:
                    cols = jax.lax.broadcasted_iota(jnp.int32, ds.shape, 1)
                    base = j * kb + c * cb + s * 128
                    ds = jnp.where(cols + base < n_keys, ds, _BIG)
                local = ds if local is None else jnp.minimum(local, ds)
        return local

    @pl.when(j == 0)
    def _():
        acc_ref[...] = jnp.full(acc_ref.shape, _BIG, jnp.float32)

    @pl.when(j < nk - 1)
    def _():
        acc_ref[...] = jnp.minimum(acc_ref[...], fold(False))

    @pl.when(j == nk - 1)
    def _():
        acc = jnp.minimum(acc_ref[...], fold(True))
        q_sq = jnp.sum(q * q, axis=1, keepdims=True)
        m = jnp.min(acc, axis=1, keepdims=True)
        o_ref[...] = jnp.sqrt(jnp.maximum(2.0 * m + q_sq, 0.0) + 1e-12)


def kernel(queries, keys):
    n_q, d_dim = queries.shape
    n_k = keys.shape[0]
    qb = 512
    kb = 2048
    nq = n_q // qb
    nk = (n_k + kb - 1) // kb
    out = pl.pallas_call(
        functools.partial(_nn_kernel, n_keys=n_k, kb=kb, nk=nk, cb=2048),
        grid=(nq, nk),
        in_specs=[
            pl.BlockSpec((qb, d_dim), lambda i, j: (i, 0)),
            pl.BlockSpec((kb, d_dim), lambda i, j: (j, 0)),
        ],
        out_specs=pl.BlockSpec((qb, 1), lambda i, j: (i, 0)),
        out_shape=jax.ShapeDtypeStruct((n_q, 1), jnp.float32),
        scratch_shapes=[pltpu.VMEM((qb, 128), jnp.float32)],
        compiler_params=pltpu.CompilerParams(
            dimension_semantics=("parallel", "arbitrary")),
    )(queries, keys)
    return out[:, 0]


# speedup vs baseline: 6.4236x; 1.0934x over previous
import functools

import jax
import jax.numpy as jnp
from jax.experimental import pallas as pl
from jax.experimental.pallas import tpu as pltpu

_BIG = 1e30


def _nn_kernel(q_ref, k_ref, o_ref, acc_ref, *, n_keys, kb, nk, cb):
    j = pl.program_id(1)
    q = q_ref[...]
    k = k_ref[...]
    qbf = q.astype(jnp.bfloat16)
    halves = jnp.full((8, q.shape[1]), 0.5, jnp.float32)

    def fold(masked):
        local = None
        for c in range(kb // cb):
            kc = k if cb == kb else k[c * cb:(c + 1) * cb]
            dots = jax.lax.dot_general(
                qbf, kc.astype(jnp.bfloat16), (((1,), (1,)), ((), ())),
                preferred_element_type=jnp.float32)
            half_ksq = jax.lax.dot_general(
                halves, kc * kc, (((1,), (1,)), ((), ())),
                preferred_element_type=jnp.float32)[:1]
            for s in range(cb // 128):
                sl = slice(s * 128, (s + 1) * 128)
                ds = half_ksq[:, sl] - dots[:, sl]
                if masked:
                    cols = jax.lax.broadcasted_iota(jnp.int32, ds.shape, 1)
                    base = j * kb + c * cb + s * 128
                    ds = jnp.where(cols + base < n_keys, ds, _BIG)
                local = ds if local is None else jnp.minimum(local, ds)
        return local

    @pl.when(j == 0)
    def _():
        acc_ref[...] = jnp.full(acc_ref.shape, _BIG, jnp.float32)

    @pl.when(j < nk - 1)
    def _():
        acc_ref[...] = jnp.minimum(acc_ref[...], fold(False))

    @pl.when(j == nk - 1)
    def _():
        acc = jnp.minimum(acc_ref[...], fold(True))
        q_sq = jnp.sum(q * q, axis=1, keepdims=True)
        m = jnp.min(acc, axis=1, keepdims=True)
        o_ref[...] = jnp.sqrt(jnp.maximum(2.0 * m + q_sq, 0.0) + 1e-12)


def kernel(queries, keys):
    n_q, d_dim = queries.shape
    n_k = keys.shape[0]
    qb = 1024
    kb = 2048
    nq = n_q // qb
    nk = (n_k + kb - 1) // kb
    out = pl.pallas_call(
        functools.partial(_nn_kernel, n_keys=n_k, kb=kb, nk=nk, cb=2048),
        grid=(nq, nk),
        in_specs=[
            pl.BlockSpec((qb, d_dim), lambda i, j: (i, 0)),
            pl.BlockSpec((kb, d_dim), lambda i, j: (j, 0)),
        ],
        out_specs=pl.BlockSpec((qb, 1), lambda i, j: (i, 0)),
        out_shape=jax.ShapeDtypeStruct((n_q, 1), jnp.float32),
        scratch_shapes=[pltpu.VMEM((qb, 128), jnp.float32)],
        compiler_params=pltpu.CompilerParams(
            dimension_semantics=("parallel", "arbitrary")),
    )(queries, keys)
    return out[:, 0]


# speedup vs baseline: 9.0933x; 1.4156x over previous
import functools

import jax
import jax.numpy as jnp
from jax.experimental import pallas as pl
from jax.experimental.pallas import tpu as pltpu

_BIG = 1e30


def _nn_kernel(q_ref, k_ref, o_ref, acc_ref, *, n_keys, kb, nk, cb):
    j = pl.program_id(1)
    q = q_ref[...]
    k = k_ref[...]
    qbf = q.astype(jnp.bfloat16)
    halves = jnp.full((8, q.shape[1]), 0.5, jnp.float32)

    local = None
    for c in range(kb // cb):
        kc = k if cb == kb else k[c * cb:(c + 1) * cb]
        dots = jax.lax.dot_general(
            qbf, kc.astype(jnp.bfloat16), (((1,), (1,)), ((), ())),
            preferred_element_type=jnp.float32)
        half_ksq = jax.lax.dot_general(
            halves, kc * kc, (((1,), (1,)), ((), ())),
            preferred_element_type=jnp.float32)[:1]
        for s in range(cb // 128):
            sl = slice(s * 128, (s + 1) * 128)
            ds = half_ksq[:, sl] - dots[:, sl]
            off = c * cb + s * 128
            if (nk - 1) * kb + off + 128 > n_keys:
                cols = jax.lax.broadcasted_iota(jnp.int32, ds.shape, 1)
                ds = jnp.where(cols + (j * kb + off) < n_keys, ds, _BIG)
            local = ds if local is None else jnp.minimum(local, ds)

    @pl.when(j == 0)
    def _():
        acc_ref[...] = local

    @pl.when(j > 0)
    def _():
        acc_ref[...] = jnp.minimum(acc_ref[...], local)

    @pl.when(j == nk - 1)
    def _():
        q_sq = jnp.sum(q * q, axis=1, keepdims=True)
        m = jnp.min(acc_ref[...], axis=1, keepdims=True)
        o_ref[...] = jnp.sqrt(jnp.maximum(2.0 * m + q_sq, 0.0) + 1e-12)


def kernel(queries, keys):
    n_q, d_dim = queries.shape
    n_k = keys.shape[0]
    qb = 1024
    kb = 2048
    nq = n_q // qb
    nk = (n_k + kb - 1) // kb
    out = pl.pallas_call(
        functools.partial(_nn_kernel, n_keys=n_k, kb=kb, nk=nk, cb=256),
        grid=(nq, nk),
        in_specs=[
            pl.BlockSpec((qb, d_dim), lambda i, j: (i, 0)),
            pl.BlockSpec((kb, d_dim), lambda i, j: (j, 0)),
        ],
        out_specs=pl.BlockSpec((qb, 1), lambda i, j: (i, 0)),
        out_shape=jax.ShapeDtypeStruct((n_q, 1), jnp.float32),
        scratch_shapes=[pltpu.VMEM((qb, 128), jnp.float32)],
        compiler_params=pltpu.CompilerParams(
            dimension_semantics=("parallel", "arbitrary")),
    )(queries, keys)
    return out[:, 0]


# speedup vs baseline: 9.8497x; 1.0832x over previous
import functools

import jax
import jax.numpy as jnp
from jax.experimental import pallas as pl
from jax.experimental.pallas import tpu as pltpu

_BIG = 1e30


def _nn_kernel(q_ref, k_ref, o_ref, acc_ref, *, n_keys, kb, nk, cb, qsub):
    j = pl.program_id(1)
    q = q_ref[...]
    k = k_ref[...]
    qb = q.shape[0]
    qbf = q.astype(jnp.bfloat16)
    kbf = k.astype(jnp.bfloat16)
    halves = jnp.full((8, q.shape[1]), 0.5, jnp.float32)
    half_ksq = jax.lax.dot_general(
        halves, k * k, (((1,), (1,)), ((), ())),
        preferred_element_type=jnp.float32)[:1]

    def run(masked):
        for t in range(qb // qsub):
            qt = qbf[t * qsub:(t + 1) * qsub]
            local = None
            for c in range(kb // cb):
                base = (nk - 1) * kb + c * cb
                if masked and base >= n_keys:
                    continue
                kc = kbf if cb == kb else kbf[c * cb:(c + 1) * cb]
                dots = jax.lax.dot_general(
                    qt, kc, (((1,), (1,)), ((), ())),
                    preferred_element_type=jnp.float32)
                for s in range(cb // 128):
                    off = c * cb + s * 128
                    gbase = (nk - 1) * kb + off
                    if masked and gbase >= n_keys:
                        continue
                    ds = half_ksq[:, off:off + 128] - \
                        dots[:, s * 128:(s + 1) * 128]
                    if masked and gbase + 128 > n_keys:
                        cols = jax.lax.broadcasted_iota(
                            jnp.int32, ds.shape, 1)
                        ds = jnp.where(cols + gbase < n_keys, ds, _BIG)
                    local = ds if local is None else jnp.minimum(local, ds)

            rows = slice(t * qsub, (t + 1) * qsub)

            @pl.when(j == 0)
            def _(local=local, rows=rows):
                acc_ref[rows, :] = local

            @pl.when(j > 0)
            def _(local=local, rows=rows):
                acc_ref[rows, :] = jnp.minimum(acc_ref[rows, :], local)

    @pl.when(j < nk - 1)
    def _():
        run(False)

    @pl.when(j == nk - 1)
    def _():
        run(True)
        q_sq = jnp.sum(q * q, axis=1, keepdims=True)
        m = jnp.min(acc_ref[...], axis=1, keepdims=True)
        o_ref[...] = jnp.sqrt(jnp.maximum(2.0 * m + q_sq, 0.0) + 1e-12)


def kernel(queries, keys):
    n_q, d_dim = queries.shape
    n_k = keys.shape[0]
    qb = 1024
    kb = 2048
    nq = n_q // qb
    nk = (n_k + kb - 1) // kb
    out = pl.pallas_call(
        functools.partial(_nn_kernel, n_keys=n_k, kb=kb, nk=nk, cb=256, qsub=1024),
        grid=(nq, nk),
        in_specs=[
            pl.BlockSpec((qb, d_dim), lambda i, j: (i, 0)),
            pl.BlockSpec((kb, d_dim), lambda i, j: (j, 0)),
        ],
        out_specs=pl.BlockSpec((qb, 1), lambda i, j: (i, 0)),
        out_shape=jax.ShapeDtypeStruct((n_q, 1), jnp.float32),
        scratch_shapes=[pltpu.VMEM((qb, 128), jnp.float32)],
        compiler_params=pltpu.CompilerParams(
            dimension_semantics=("parallel", "arbitrary")),
    )(queries, keys)
    return out[:, 0]


# speedup vs baseline: 11.6908x; 1.1869x over previous
import functools

import jax
import jax.numpy as jnp
from jax.experimental import pallas as pl
from jax.experimental.pallas import tpu as pltpu

_BIG = 1e30


def _nn_kernel(q_ref, k_ref, o_ref, acc_ref, *, n_keys, kb, nk, cb, qsub):
    j = pl.program_id(1)
    q = q_ref[...]
    k = k_ref[...]
    qb = q.shape[0]
    qbf = q.astype(jnp.bfloat16)
    kbf = k.astype(jnp.bfloat16)
    halves = jnp.full((8, q.shape[1]), 0.5, jnp.float32)
    half_ksq = jax.lax.dot_general(
        halves, k * k, (((1,), (1,)), ((), ())),
        preferred_element_type=jnp.float32)[:1]

    for t in range(qb // qsub):
        qt = qbf[t * qsub:(t + 1) * qsub]
        local = None
        for c in range(kb // cb):
            kc = kbf if cb == kb else kbf[c * cb:(c + 1) * cb]
            dots = jax.lax.dot_general(
                qt, kc, (((1,), (1,)), ((), ())),
                preferred_element_type=jnp.float32)
            for s in range(cb // 128):
                off = c * cb + s * 128
                sl = slice(off, off + 128)
                ds = half_ksq[:, sl] - dots[:, s * 128:(s + 1) * 128]
                if (nk - 1) * kb + off + 128 > n_keys:
                    cols = jax.lax.broadcasted_iota(jnp.int32, ds.shape, 1)
                    ds = jnp.where(cols + (j * kb + off) < n_keys, ds, _BIG)
                local = ds if local is None else jnp.minimum(local, ds)

        rows = slice(t * qsub, (t + 1) * qsub)

        @pl.when(j == 0)
        def _(local=local, rows=rows):
            acc_ref[rows, :] = local

        @pl.when(j > 0)
        def _(local=local, rows=rows):
            acc_ref[rows, :] = jnp.minimum(acc_ref[rows, :], local)

    @pl.when(j == nk - 1)
    def _():
        q_sq = jnp.sum(q * q, axis=1, keepdims=True)
        m = jnp.min(acc_ref[...], axis=1, keepdims=True)
        o_ref[...] = jnp.sqrt(jnp.maximum(2.0 * m + q_sq, 0.0) + 1e-12)


def kernel(queries, keys):
    n_q, d_dim = queries.shape
    n_k = keys.shape[0]
    qb = 1024
    kb = 8192
    nq = n_q // qb
    nk = (n_k + kb - 1) // kb
    out = pl.pallas_call(
        functools.partial(_nn_kernel, n_keys=n_k, kb=kb, nk=nk, cb=256, qsub=1024),
        grid=(nq, nk),
        in_specs=[
            pl.BlockSpec((qb, d_dim), lambda i, j: (i, 0)),
            pl.BlockSpec((kb, d_dim), lambda i, j: (j, 0)),
        ],
        out_specs=pl.BlockSpec((qb, 1), lambda i, j: (i, 0)),
        out_shape=jax.ShapeDtypeStruct((n_q, 1), jnp.float32),
        scratch_shapes=[pltpu.VMEM((qb, 128), jnp.float32)],
        compiler_params=pltpu.CompilerParams(
            dimension_semantics=("parallel", "arbitrary")),
    )(queries, keys)
    return out[:, 0]


# speedup vs baseline: 12.0456x; 1.0304x over previous
import functools

import jax
import jax.numpy as jnp
from jax.experimental import pallas as pl
from jax.experimental.pallas import tpu as pltpu

_PAD_VAL = 1e4


def _min_fold(qbf, kbf, half_ksq, local, cb):
    kb = kbf.shape[0]
    for c in range(kb // cb):
        kc = kbf if cb == kb else kbf[c * cb:(c + 1) * cb]
        dots = jax.lax.dot_general(
            qbf, kc, (((1,), (1,)), ((), ())),
            preferred_element_type=jnp.float32)
        for s in range(cb // 128):
            off = c * cb + s * 128
            ds = half_ksq[:, off:off + 128] - dots[:, s * 128:(s + 1) * 128]
            local = ds if local is None else jnp.minimum(local, ds)
    return local


def _half_ksq_row(k):
    halves = jnp.full((8, k.shape[1]), 0.5, jnp.float32)
    return jax.lax.dot_general(
        halves, k * k, (((1,), (1,)), ((), ())),
        preferred_element_type=jnp.float32)[:1]


def _nn_kernel(q_ref, k_ref, t_ref, o_ref, acc_ref, *, nk, cb):
    j = pl.program_id(1)
    q = q_ref[...]
    k = k_ref[...]
    qbf = q.astype(jnp.bfloat16)

    local = _min_fold(qbf, k.astype(jnp.bfloat16), _half_ksq_row(k),
                      None, cb)

    @pl.when(j == 0)
    def _():
        acc_ref[...] = local

    @pl.when(j > 0)
    def _():
        acc_ref[...] = jnp.minimum(acc_ref[...], local)

    @pl.when(j == nk - 1)
    def _():
        t = t_ref[...]
        acc = _min_fold(qbf, t.astype(jnp.bfloat16), _half_ksq_row(t),
                        acc_ref[...], t.shape[0])
        q_sq = jnp.sum(q * q, axis=1, keepdims=True)
        m = jnp.min(acc, axis=1, keepdims=True)
        o_ref[...] = jnp.sqrt(jnp.maximum(2.0 * m + q_sq, 0.0) + 1e-12)


def kernel(queries, keys):
    n_q, d_dim = queries.shape
    n_k = keys.shape[0]
    qb = 1024
    kb = 8192
    cb = 256
    nq = n_q // qb
    nk = n_k // kb
    n_tail = n_k - nk * kb
    tb = max(-(-n_tail // 128) * 128, 128)
    tail = jnp.pad(keys[nk * kb:], ((0, tb - n_tail), (0, 0)),
                   constant_values=_PAD_VAL)
    out = pl.pallas_call(
        functools.partial(_nn_kernel, nk=nk, cb=cb),
        grid=(nq, nk),
        in_specs=[
            pl.BlockSpec((qb, d_dim), lambda i, j: (i, 0)),
            pl.BlockSpec((kb, d_dim), lambda i, j: (j, 0)),
            pl.BlockSpec((tb, d_dim), lambda i, j: (0, 0)),
        ],
        out_specs=pl.BlockSpec((qb, 1), lambda i, j: (i, 0)),
        out_shape=jax.ShapeDtypeStruct((n_q, 1), jnp.float32),
        scratch_shapes=[pltpu.VMEM((qb, 128), jnp.float32)],
        compiler_params=pltpu.CompilerParams(
            dimension_semantics=("parallel", "arbitrary")),
    )(queries, keys, tail)
    return out[:, 0]
